# initial kernel scaffold (unmeasured)
import jax
import jax.numpy as jnp
from jax import lax
from jax.experimental import pallas as pl
from jax.experimental.pallas import tpu as pltpu

N_DEV = 32


def kernel(x, Win0, Wout0, Win1, Wout1, Win2, Wout2):
    m_per, d_model = x.shape
    m_full = N_DEV * m_per

    def body(x_ref, win0, wout0, win1, wout1, win2, wout2, out_ref,
             xfull, partial, comm, ag_send, ag_recv, rs_send, rs_recv):
        my = lax.axis_index("i")
        right = lax.rem(my + 1, N_DEV)

        def ag_phase():
            def hop(h, _):
                origin = lax.rem(my - h + N_DEV, N_DEV)
                rdma = pltpu.make_async_remote_copy(
                    src_ref=xfull.at[pl.ds(origin * m_per, m_per), :],
                    dst_ref=xfull.at[pl.ds(origin * m_per, m_per), :],
                    send_sem=ag_send.at[h],
                    recv_sem=ag_recv.at[h],
                    device_id=(right,),
                    device_id_type=pl.DeviceIdType.MESH,
                )
                rdma.start()
                rdma.wait()
                return 0
            lax.fori_loop(0, N_DEV - 1, hop, 0)

        def compute_layer(win, wout):
            n_blk = 8
            rows = m_full // n_blk
            for b in range(n_blk):
                xb = xfull[pl.ds(b * rows, rows), :]
                hb = jnp.maximum(
                    jnp.dot(xb, win[...], preferred_element_type=jnp.float32),
                    0.0)
                partial[pl.ds(b * rows, rows), :] = jnp.dot(
                    hb, wout[...], preferred_element_type=jnp.float32)

        def rs_phase():
            c0 = lax.rem(my - 1 + N_DEV, N_DEV)
            comm[pl.ds(0, m_per), :] = partial[pl.ds(c0 * m_per, m_per), :]

            def hop(s, _):
                rdma = pltpu.make_async_remote_copy(
                    src_ref=comm.at[pl.ds(s * m_per, m_per), :],
                    dst_ref=comm.at[pl.ds((s + 1) * m_per, m_per), :],
                    send_sem=rs_send.at[s],
                    recv_sem=rs_recv.at[s],
                    device_id=(right,),
                    device_id_type=pl.DeviceIdType.MESH,
                )
                rdma.start()
                rdma.wait()
                c = lax.rem(my - 2 - s + 2 * N_DEV, N_DEV)
                comm[pl.ds((s + 1) * m_per, m_per), :] = (
                    comm[pl.ds((s + 1) * m_per, m_per), :]
                    + partial[pl.ds(c * m_per, m_per), :])
                return 0
            lax.fori_loop(0, N_DEV - 1, hop, 0)

        last = pl.ds((N_DEV - 1) * m_per, m_per)
        mine = pl.ds(my * m_per, m_per)

        xfull[mine, :] = x_ref[...]
        ag_phase()

        compute_layer(win0, wout0)
        rs_phase()
        xfull[mine, :] = comm[last, :]
        ag_phase()

        compute_layer(win1, wout1)
        rs_phase()
        xfull[mine, :] = comm[last, :]
        ag_phase()

        compute_layer(win2, wout2)
        rs_phase()
        out_ref[...] = comm[last, :]

    return pl.pallas_call(
        body,
        out_shape=jax.ShapeDtypeStruct((m_per, d_model), jnp.float32),
        in_specs=[pl.BlockSpec(memory_space=pltpu.VMEM)] * 7,
        out_specs=pl.BlockSpec(memory_space=pltpu.VMEM),
        scratch_shapes=[
            pltpu.VMEM((m_full, d_model), jnp.float32),
            pltpu.VMEM((m_full, d_model), jnp.float32),
            pltpu.VMEM((m_full, d_model), jnp.float32),
            pltpu.SemaphoreType.DMA((N_DEV,)),
            pltpu.SemaphoreType.DMA((N_DEV,)),
            pltpu.SemaphoreType.DMA((N_DEV,)),
            pltpu.SemaphoreType.DMA((N_DEV,)),
        ],
        compiler_params=pltpu.CompilerParams(collective_id=0),
    )(x, Win0, Wout0, Win1, Wout1, Win2, Wout2)


# baseline (device time: 648662 ns/iter reference)
import jax
import jax.numpy as jnp
from jax import lax
from jax.experimental import pallas as pl
from jax.experimental.pallas import tpu as pltpu

N_DEV = 32


def kernel(x, Win0, Wout0, Win1, Wout1, Win2, Wout2):
    m_per, d_model = x.shape
    m_full = N_DEV * m_per

    def body(x_ref, win0, wout0, win1, wout1, win2, wout2, out_ref,
             xfull, partial, comm, ag_send, ag_recv, rs_send, rs_recv):
        my = lax.axis_index("i")
        right = lax.rem(my + 1, N_DEV)

        def ag_phase():
            def hop(h, _):
                origin = lax.rem(my - h + N_DEV, N_DEV)
                rdma = pltpu.make_async_remote_copy(
                    src_ref=xfull.at[pl.ds(origin * m_per, m_per), :],
                    dst_ref=xfull.at[pl.ds(origin * m_per, m_per), :],
                    send_sem=ag_send.at[h],
                    recv_sem=ag_recv.at[h],
                    device_id=(right,),
                    device_id_type=pl.DeviceIdType.MESH,
                )
                rdma.start()
                rdma.wait()
                return 0
            lax.fori_loop(0, N_DEV - 1, hop, 0)

        def compute_layer(win, wout):
            n_blk = 8
            rows = m_full // n_blk
            for b in range(n_blk):
                xb = xfull[pl.ds(b * rows, rows), :]
                hb = jnp.maximum(
                    jnp.dot(xb, win[...], preferred_element_type=jnp.float32),
                    0.0)
                partial[pl.ds(b * rows, rows), :] = jnp.dot(
                    hb, wout[...], preferred_element_type=jnp.float32)

        def rs_phase():
            c0 = lax.rem(my - 1 + N_DEV, N_DEV)
            comm[pl.ds(0, m_per), :] = partial[pl.ds(c0 * m_per, m_per), :]

            def hop(s, _):
                rdma = pltpu.make_async_remote_copy(
                    src_ref=comm.at[pl.ds(s * m_per, m_per), :],
                    dst_ref=comm.at[pl.ds((s + 1) * m_per, m_per), :],
                    send_sem=rs_send.at[s],
                    recv_sem=rs_recv.at[s],
                    device_id=(right,),
                    device_id_type=pl.DeviceIdType.MESH,
                )
                rdma.start()
                rdma.wait()
                c = lax.rem(my - 2 - s + 2 * N_DEV, N_DEV)
                comm[pl.ds((s + 1) * m_per, m_per), :] = (
                    comm[pl.ds((s + 1) * m_per, m_per), :]
                    + partial[pl.ds(c * m_per, m_per), :])
                return 0
            lax.fori_loop(0, N_DEV - 1, hop, 0)

        last = pl.ds((N_DEV - 1) * m_per, m_per)
        mine = pl.ds(my * m_per, m_per)

        xfull[mine, :] = x_ref[...]
        ag_phase()

        compute_layer(win0, wout0)
        rs_phase()
        xfull[mine, :] = comm[last, :]
        ag_phase()

        compute_layer(win1, wout1)
        rs_phase()
        xfull[mine, :] = comm[last, :]
        ag_phase()

        compute_layer(win2, wout2)
        rs_phase()
        out_ref[...] = comm[last, :]

    return pl.pallas_call(
        body,
        out_shape=jax.ShapeDtypeStruct((m_per, d_model), jnp.float32),
        in_specs=[pl.BlockSpec(memory_space=pltpu.VMEM)] * 7,
        out_specs=pl.BlockSpec(memory_space=pltpu.VMEM),
        scratch_shapes=[
            pltpu.VMEM((m_full, d_model), jnp.float32),
            pltpu.VMEM((m_full, d_model), jnp.float32),
            pltpu.VMEM((m_full, d_model), jnp.float32),
            pltpu.SemaphoreType.DMA((N_DEV,)),
            pltpu.SemaphoreType.DMA((N_DEV,)),
            pltpu.SemaphoreType.DMA((N_DEV,)),
            pltpu.SemaphoreType.DMA((N_DEV,)),
        ],
    )(x, Win0, Wout0, Win1, Wout1, Win2, Wout2)
